# baseline (speedup 1.0000x reference)
import numpy as np

import jax
import jax.numpy as jnp
from jax.experimental import pallas as pl

_BB = 16

_C1W, _C2W, _G1W, _G2W, _FCW = 0, 720, 3280, 5328, 9424
_B1, _B2, _G1B, _G2B, _FCB, _ZERO = 13520, 13536, 13568, 13632, 13696, 13760


def _param_index() -> np.ndarray:
    idx = np.full((64, 1280), _ZERO, np.int64)
    for p_out in range(4):
        for k in range(5):
            r = p_out + k - 2
            cols = (r // 4 + 1) * 36 + np.arange(9) * 4 + r % 4
            rows = np.arange(16) + 16 * p_out
            idx[rows[:, None], cols[None, :]] = _C1W + (
                (np.arange(16)[:, None] * 9 + np.arange(9)[None, :]) * 5 + k)
    for j in range(2):
        for k in range(5):
            r = j + k - 2
            cols = 128 + (r // 2 + 1) * 32 + (r % 2) * 16 + np.arange(16)
            rows = np.arange(32) + 32 * j
            idx[rows[:, None], cols[None, :]] = _C2W + (
                (np.arange(32)[:, None] * 16 + np.arange(16)[None, :]) * 5 + k)
    idx[:, 256:288] = _G1W + np.arange(32)[None, :] * 64 + np.arange(64)[:, None]
    idx[:, 384:448] = _G2W + np.arange(64)[None, :] * 64 + np.arange(64)[:, None]
    idx[:, 512:576] = _FCW + np.arange(64)[None, :] * 64 + np.arange(64)[:, None]
    idx[:, 640] = _B1 + np.arange(64) % 16
    idx[:, 768] = _B2 + np.arange(64) % 32
    idx[:, 896] = _G1B + np.arange(64)
    idx[:, 1024] = _G2B + np.arange(64)
    idx[:, 1152] = _FCB + np.arange(64)
    return idx


_PIDX = _param_index()


def _fused_kernel(x_ref, wp_ref, out_ref):
    tq = x_ref.shape[1]
    bb = out_ref.shape[0]
    lb = bb * tq

    wp = wp_ref[...]
    w1b, w2b = wp[:, 0:108], wp[:, 128:224]
    g1w, g2w, fcw = wp[:, 256:288], wp[:, 384:448], wp[:, 512:576]
    b1, b2 = wp[:, 640:641], wp[:, 768:769]
    g1b, g2b, fcb = wp[:, 896:897], wp[:, 1024:1025], wp[:, 1152:1153]

    li = jax.lax.broadcasted_iota(jnp.int32, (1, lb), 1) % tq
    first = li == 0
    last = li == tq - 1

    def shifts(a):
        z = jnp.zeros_like(a[:, :1])
        plus = jnp.where(last, 0.0, jnp.concatenate([a[:, 1:], z], 1))
        minus = jnp.where(first, 0.0, jnp.concatenate([z, a[:, :-1]], 1))
        return minus, plus

    def mm(a, b):
        return jax.lax.dot_general(a, b, (((1,), (0,)), ((), ())),
                                   preferred_element_type=jnp.float32)

    xr = x_ref[...]
    nrow = xr.shape[0] // bb
    xb = jnp.concatenate(
        [xr[s * nrow:(s + 1) * nrow, :] for s in range(bb)], axis=1)

    m1, p1 = shifts(xb)
    h = jnp.maximum(mm(w1b, jnp.concatenate([m1, xb, p1], 0)) + b1, 0.0)
    pe = jnp.maximum(h[0:16], h[16:32])
    po = jnp.maximum(h[32:48], h[48:64])
    h1 = jnp.concatenate([pe, po], axis=0)

    m2, p2 = shifts(h1)
    h = jnp.maximum(mm(w2b, jnp.concatenate([m2, h1, p2], 0)) + b2, 0.0)
    nodes = jnp.maximum(h[0:32], h[32:64])

    dinv = jnp.where(first | last, jax.lax.rsqrt(2.0), jax.lax.rsqrt(3.0))

    def gcn(n, w, b):
        g = mm(w, n) * dinv
        gm, gp = shifts(g)
        return jnp.maximum((g + gm + gp) * dinv + b, 0.0)

    nodes = gcn(nodes, g1w, g1b)
    nodes = gcn(nodes, g2w, g2b)

    pooled = jnp.sum(nodes.reshape(64, lb // tq, tq), axis=2) * (1.0 / tq)
    out = mm(fcw, pooled) + fcb
    out_ref[...] = out.T


@jax.jit
def kernel(x, conv1_w, conv1_b, conv2_w, conv2_b, gcn1_w, gcn1_b, gcn2_w,
           gcn2_b, fc_w, fc_b):
    b, c_in, t_in = x.shape
    tq = t_in // 4
    out_f = fc_w.shape[1]

    xr = x.reshape(b, c_in, tq, 4).transpose(0, 1, 3, 2).reshape(
        b * c_in * 4, tq)

    flat = jnp.concatenate([
        conv1_w.reshape(-1), conv2_w.reshape(-1), gcn1_w.reshape(-1),
        gcn2_w.reshape(-1), fc_w.reshape(-1), conv1_b, conv2_b, gcn1_b,
        gcn2_b, fc_b, jnp.zeros(1, jnp.float32)])
    wpack = flat[_PIDX]

    return pl.pallas_call(
        _fused_kernel,
        grid=(b // _BB,),
        in_specs=[pl.BlockSpec((_BB * c_in * 4, tq), lambda i: (i, 0)),
                  pl.BlockSpec(wpack.shape, lambda i: (0, 0))],
        out_specs=pl.BlockSpec((_BB, out_f), lambda i: (i, 0)),
        out_shape=jax.ShapeDtypeStruct((b, out_f), x.dtype),
    )(xr, wpack)
